# baseline (speedup 1.0000x reference)
import functools

import jax
import jax.numpy as jnp
from jax import lax
from jax.experimental import pallas as pl
from jax.experimental.pallas import tpu as pltpu
from jax.experimental.pallas import tpu_sc as plsc


def _gate_body(xs_ref, xt_ref, wc_ref, wt_ref, bias_ref, mask_ref, wsq_ref,
               bsq_ref, a_ref, c_ref):
    dn = (((1,), (1,)), ((), ()))
    xs = xs_ref[...].astype(jnp.bfloat16)
    xt = xt_ref[...].astype(jnp.bfloat16)
    pre = (lax.dot_general(xs, wc_ref[...], dn,
                           preferred_element_type=jnp.float32)
           + lax.dot_general(xt, wt_ref[...], dn,
                             preferred_element_type=jnp.float32))
    t2 = jnp.tanh(pre + bias_ref[...])
    logit = lax.dot_general(t2, wsq_ref[...], dn,
                            preferred_element_type=jnp.float32)
    mask = mask_ref[...]
    a = jax.nn.sigmoid(logit * mask + bsq_ref[0]) * mask
    a_ref[...] = a
    c_ref[...] = mask - a


def _compute_gate(xs, xt, wc, wt, bias, mask2, wsq, bsq):
    rows, d_in = xs.shape
    d_h = wc.shape[0]
    blk_r = 128
    grid = (rows // blk_r,)
    return pl.pallas_call(
        _gate_body,
        grid=grid,
        in_specs=[
            pl.BlockSpec((blk_r, d_in), lambda r: (r, 0)),
            pl.BlockSpec((blk_r, d_in), lambda r: (r, 0)),
            pl.BlockSpec((d_h, d_in), lambda r: (0, 0)),
            pl.BlockSpec((d_h, d_in), lambda r: (0, 0)),
            pl.BlockSpec((1, d_h), lambda r: (0, 0)),
            pl.BlockSpec((blk_r, 1), lambda r: (r, 0)),
            pl.BlockSpec((1, d_h), lambda r: (0, 0)),
            pl.BlockSpec(memory_space=pltpu.SMEM),
        ],
        out_specs=[
            pl.BlockSpec((blk_r, 1), lambda r: (r, 0)),
            pl.BlockSpec((blk_r, 1), lambda r: (r, 0)),
        ],
        out_shape=[
            jax.ShapeDtypeStruct((rows, 1), jnp.float32),
            jax.ShapeDtypeStruct((rows, 1), jnp.float32),
        ],
    )(xs, xt, wc, wt, bias, mask2, wsq, bsq)


_LANES = 16
_NBUF = 4


def _scatter_body(t_rows, s_pad, vocab,
                  tsa_hbm, smask_hbm, idx_hbm, fs_hbm,
                  idx_v, smask_v, tsa_v, b0, b1, b2, b3, s0, s1, s2, s3):
    cid = lax.axis_index("c")
    sid = lax.axis_index("s")
    b = sid * 2 + cid
    row0 = b * t_rows

    pltpu.sync_copy(idx_hbm.at[b], idx_v)
    pltpu.sync_copy(smask_hbm.at[b], smask_v)
    pltpu.sync_copy(tsa_hbm.at[b], tsa_v)

    n_chunks = s_pad // _LANES
    bufs = (b0, b1, b2, b3)
    sems = (s0, s1, s2, s3)
    zero = jnp.zeros((_LANES,), jnp.float32)

    for i in range(_NBUF):
        buf = bufs[i]

        @plsc.parallel_loop(0, vocab, step=_LANES, unroll=8)
        def _(j):
            buf[pl.ds(j, _LANES)] = zero

    @pl.loop(0, t_rows, step=_NBUF)
    def _(g):
        for i in range(_NBUF):
            t = g + i
            buf, sem = bufs[i], sems[i]
            @pl.when(g >= _NBUF)
            def _():
                pltpu.make_async_copy(buf, fs_hbm.at[pl.ds(0, vocab)],
                                      sem).wait()
            for jc in range(n_chunks):
                cols = idx_v[pl.ds(jc * _LANES, _LANES)]
                plsc.store_scatter(buf, [cols], zero)
            for jc in range(n_chunks):
                cols = idx_v[pl.ds(jc * _LANES, _LANES)]
                val = (tsa_v[pl.ds(t * s_pad + jc * _LANES, _LANES)]
                       * smask_v[pl.ds(jc * _LANES, _LANES)])
                plsc.addupdate_scatter(buf, [cols], val)
            pltpu.async_copy(buf, fs_hbm.at[pl.ds((row0 + t) * vocab, vocab)],
                             sem)

    for i in range(_NBUF):
        pltpu.make_async_copy(bufs[i], fs_hbm.at[pl.ds(0, vocab)],
                              sems[i]).wait()


def _scatter_fs(tsa_p, smask_p, idx_p, n_rows, vocab):
    n_batch, t_rows, s_pad = tsa_p.shape
    mesh = plsc.VectorSubcoreMesh(core_axis_name="c", subcore_axis_name="s",
                                  num_cores=2, num_subcores=16)
    body = functools.partial(_scatter_body, t_rows, s_pad, vocab)
    return pl.kernel(
        body,
        out_type=jax.ShapeDtypeStruct((n_rows * vocab,), jnp.float32),
        mesh=mesh,
        compiler_params=pltpu.CompilerParams(needs_layout_passes=False),
        scratch_types=[
            pltpu.VMEM((s_pad,), jnp.int32),
            pltpu.VMEM((s_pad,), jnp.float32),
            pltpu.VMEM((t_rows * s_pad,), jnp.float32),
            pltpu.VMEM((vocab,), jnp.float32),
            pltpu.VMEM((vocab,), jnp.float32),
            pltpu.VMEM((vocab,), jnp.float32),
            pltpu.VMEM((vocab,), jnp.float32),
            pltpu.SemaphoreType.DMA,
            pltpu.SemaphoreType.DMA,
            pltpu.SemaphoreType.DMA,
            pltpu.SemaphoreType.DMA,
        ],
    )(tsa_p.reshape(n_batch, t_rows * s_pad), smask_p, idx_p)


def _fmerge_body(trg_ref, fs_ref, a_ref, c_ref, out_ref):
    out_ref[...] = (a_ref[...] * trg_ref[...]
                    + c_ref[...] * fs_ref[...])


def _final_merge(trg2, fs2, a2, c2):
    n_rows, vocab = trg2.shape
    blk_r = 128
    grid = (n_rows // blk_r,)
    return pl.pallas_call(
        _fmerge_body,
        grid=grid,
        in_specs=[
            pl.BlockSpec((blk_r, vocab), lambda r: (r, 0)),
            pl.BlockSpec((blk_r, vocab), lambda r: (r, 0)),
            pl.BlockSpec((blk_r, 1), lambda r: (r, 0)),
            pl.BlockSpec((blk_r, 1), lambda r: (r, 0)),
        ],
        out_specs=pl.BlockSpec((blk_r, vocab), lambda r: (r, 0)),
        out_shape=jax.ShapeDtypeStruct((n_rows, vocab), jnp.float32),
    )(trg2, fs2, a2, c2)


def kernel(target_target_representations, target_source_representations,
           trg_decoder_output, target_mask, target_source_attention,
           source_mask, input_source, W_ctx, b_ctx, W_tgt, b_tgt, W_sq, b_sq):
    batch, t_rows, d_in = target_target_representations.shape
    vocab = trg_decoder_output.shape[-1]
    s_len = target_source_attention.shape[-1]
    rows = batch * t_rows

    xs = target_source_representations.reshape(rows, d_in)
    xt = target_target_representations.reshape(rows, d_in)
    wc = W_ctx.astype(jnp.bfloat16)
    wt = W_tgt.astype(jnp.bfloat16)
    bias = (b_ctx + b_tgt).reshape(1, -1)
    mask2 = target_mask.reshape(rows, 1)
    bsq = b_sq.reshape(1)

    a2, c2 = _compute_gate(xs, xt, wc, wt, bias, mask2, W_sq, bsq)

    s_pad = ((s_len + _LANES - 1) // _LANES) * _LANES
    pad = s_pad - s_len
    tsa_p = jnp.pad(target_source_attention, ((0, 0), (0, 0), (0, pad)))
    smask_p = jnp.pad(source_mask, ((0, 0), (0, pad)))
    idx_p = jnp.pad(input_source.astype(jnp.int32), ((0, 0), (0, pad)))
    trg2 = trg_decoder_output.reshape(rows, vocab)

    fs_lin = _scatter_fs(tsa_p, smask_p, idx_p, rows, vocab)
    out2 = _final_merge(trg2, fs_lin.reshape(rows, vocab), a2, c2)
    return out2.reshape(batch, t_rows, vocab)


# speedup vs baseline: 1.2544x; 1.2544x over previous
import functools

import jax
import jax.numpy as jnp
from jax import lax
from jax.experimental import pallas as pl
from jax.experimental.pallas import tpu as pltpu
from jax.experimental.pallas import tpu_sc as plsc


def _gate_body(xs_ref, xt_ref, wc_ref, wt_ref, bias_ref, mask_ref, wsq_ref,
               bsq_ref, a_ref, c_ref):
    dn = (((1,), (1,)), ((), ()))
    xs = xs_ref[...].astype(jnp.bfloat16)
    xt = xt_ref[...].astype(jnp.bfloat16)
    pre = (lax.dot_general(xs, wc_ref[...], dn,
                           preferred_element_type=jnp.float32)
           + lax.dot_general(xt, wt_ref[...], dn,
                             preferred_element_type=jnp.float32))
    t2 = jnp.tanh(pre + bias_ref[...])
    logit = lax.dot_general(t2, wsq_ref[...], dn,
                            preferred_element_type=jnp.float32)
    mask = mask_ref[...]
    a = jax.nn.sigmoid(logit * mask + bsq_ref[0]) * mask
    a_ref[...] = a
    c_ref[...] = mask - a


def _compute_gate(xs, xt, wc, wt, bias, mask2, wsq, bsq):
    rows, d_in = xs.shape
    d_h = wc.shape[0]
    blk_r = 512
    grid = (rows // blk_r,)
    return pl.pallas_call(
        _gate_body,
        grid=grid,
        in_specs=[
            pl.BlockSpec((blk_r, d_in), lambda r: (r, 0)),
            pl.BlockSpec((blk_r, d_in), lambda r: (r, 0)),
            pl.BlockSpec((d_h, d_in), lambda r: (0, 0)),
            pl.BlockSpec((d_h, d_in), lambda r: (0, 0)),
            pl.BlockSpec((1, d_h), lambda r: (0, 0)),
            pl.BlockSpec((blk_r, 1), lambda r: (r, 0)),
            pl.BlockSpec((1, d_h), lambda r: (0, 0)),
            pl.BlockSpec(memory_space=pltpu.SMEM),
        ],
        out_specs=[
            pl.BlockSpec((blk_r, 1), lambda r: (r, 0)),
            pl.BlockSpec((blk_r, 1), lambda r: (r, 0)),
        ],
        out_shape=[
            jax.ShapeDtypeStruct((rows, 1), jnp.float32),
            jax.ShapeDtypeStruct((rows, 1), jnp.float32),
        ],
    )(xs, xt, wc, wt, bias, mask2, wsq, bsq)


_LANES = 16
_NBUF = 4


def _scatter_body(t_rows, s_pad, vocab,
                  tsa_hbm, smask_hbm, idx_hbm, fs_hbm,
                  idx_v, smask_v, tsa_v, b0, b1, b2, b3, s0, s1, s2, s3):
    cid = lax.axis_index("c")
    sid = lax.axis_index("s")
    b = sid * 2 + cid
    row0 = b * t_rows

    pltpu.sync_copy(idx_hbm.at[b], idx_v)
    pltpu.sync_copy(smask_hbm.at[b], smask_v)
    pltpu.sync_copy(tsa_hbm.at[b], tsa_v)

    n_chunks = s_pad // _LANES
    bufs = (b0, b1, b2, b3)
    sems = (s0, s1, s2, s3)
    zero = jnp.zeros((_LANES,), jnp.float32)

    for i in range(_NBUF):
        buf = bufs[i]

        @plsc.parallel_loop(0, vocab, step=_LANES, unroll=8)
        def _(j):
            buf[pl.ds(j, _LANES)] = zero

    @pl.loop(0, t_rows, step=_NBUF)
    def _(g):
        for i in range(_NBUF):
            t = g + i
            buf, sem = bufs[i], sems[i]
            @pl.when(g >= _NBUF)
            def _():
                pltpu.make_async_copy(buf, fs_hbm.at[pl.ds(0, vocab)],
                                      sem).wait()
            for jc in range(n_chunks):
                cols = idx_v[pl.ds(jc * _LANES, _LANES)]
                plsc.store_scatter(buf, [cols], zero)
            for jc in range(n_chunks):
                cols = idx_v[pl.ds(jc * _LANES, _LANES)]
                val = (tsa_v[pl.ds(t * s_pad + jc * _LANES, _LANES)]
                       * smask_v[pl.ds(jc * _LANES, _LANES)])
                plsc.addupdate_scatter(buf, [cols], val)
            pltpu.async_copy(buf, fs_hbm.at[pl.ds((row0 + t) * vocab, vocab)],
                             sem)

    for i in range(_NBUF):
        pltpu.make_async_copy(bufs[i], fs_hbm.at[pl.ds(0, vocab)],
                              sems[i]).wait()


def _scatter_fs(tsa_p, smask_p, idx_p, n_rows, vocab):
    n_batch, t_rows, s_pad = tsa_p.shape
    mesh = plsc.VectorSubcoreMesh(core_axis_name="c", subcore_axis_name="s",
                                  num_cores=2, num_subcores=16)
    body = functools.partial(_scatter_body, t_rows, s_pad, vocab)
    return pl.kernel(
        body,
        out_type=jax.ShapeDtypeStruct((n_rows * vocab,), jnp.float32),
        mesh=mesh,
        compiler_params=pltpu.CompilerParams(needs_layout_passes=False),
        scratch_types=[
            pltpu.VMEM((s_pad,), jnp.int32),
            pltpu.VMEM((s_pad,), jnp.float32),
            pltpu.VMEM((t_rows * s_pad,), jnp.float32),
            pltpu.VMEM((vocab,), jnp.float32),
            pltpu.VMEM((vocab,), jnp.float32),
            pltpu.VMEM((vocab,), jnp.float32),
            pltpu.VMEM((vocab,), jnp.float32),
            pltpu.SemaphoreType.DMA,
            pltpu.SemaphoreType.DMA,
            pltpu.SemaphoreType.DMA,
            pltpu.SemaphoreType.DMA,
        ],
    )(tsa_p.reshape(n_batch, t_rows * s_pad), smask_p, idx_p)


def _fmerge_body(trg_ref, fs_ref, a_ref, c_ref, out_ref):
    out_ref[...] = (a_ref[...] * trg_ref[...]
                    + c_ref[...] * fs_ref[...])


def _final_merge(trg2, fs2, a2, c2):
    n_rows, vocab = trg2.shape
    blk_r = 128
    grid = (n_rows // blk_r,)
    return pl.pallas_call(
        _fmerge_body,
        grid=grid,
        in_specs=[
            pl.BlockSpec((blk_r, vocab), lambda r: (r, 0)),
            pl.BlockSpec((blk_r, vocab), lambda r: (r, 0)),
            pl.BlockSpec((blk_r, 1), lambda r: (r, 0)),
            pl.BlockSpec((blk_r, 1), lambda r: (r, 0)),
        ],
        out_specs=pl.BlockSpec((blk_r, vocab), lambda r: (r, 0)),
        out_shape=jax.ShapeDtypeStruct((n_rows, vocab), jnp.float32),
    )(trg2, fs2, a2, c2)


def kernel(target_target_representations, target_source_representations,
           trg_decoder_output, target_mask, target_source_attention,
           source_mask, input_source, W_ctx, b_ctx, W_tgt, b_tgt, W_sq, b_sq):
    batch, t_rows, d_in = target_target_representations.shape
    vocab = trg_decoder_output.shape[-1]
    s_len = target_source_attention.shape[-1]
    rows = batch * t_rows

    xs = target_source_representations.reshape(rows, d_in)
    xt = target_target_representations.reshape(rows, d_in)
    wc = W_ctx.astype(jnp.bfloat16)
    wt = W_tgt.astype(jnp.bfloat16)
    bias = (b_ctx + b_tgt).reshape(1, -1)
    mask2 = target_mask.reshape(rows, 1)
    bsq = b_sq.reshape(1)

    a2, c2 = _compute_gate(xs, xt, wc, wt, bias, mask2, W_sq, bsq)

    s_pad = ((s_len + _LANES - 1) // _LANES) * _LANES
    pad = s_pad - s_len
    tsa_p = jnp.pad(target_source_attention, ((0, 0), (0, 0), (0, pad)))
    smask_p = jnp.pad(source_mask, ((0, 0), (0, pad)))
    idx_p = jnp.pad(input_source.astype(jnp.int32), ((0, 0), (0, pad)))
    trg2 = trg_decoder_output.reshape(rows, vocab)

    fs_lin = _scatter_fs(tsa_p, smask_p, idx_p, rows, vocab)
    out2 = _final_merge(trg2, fs_lin.reshape(rows, vocab), a2, c2)
    return out2.reshape(batch, t_rows, vocab)
